# baseline (speedup 1.0000x reference)
import functools

import jax
import jax.numpy as jnp
from jax import lax
from jax.experimental import pallas as pl
from jax.experimental.pallas import tpu as pltpu
from jax.experimental.pallas import tpu_sc as plsc

_B, _L, _N, _H, _P = 32, 20, 50000, 512, 256
_NBLK = 2048


def _stats_kernel(likes_ref, fmo_ref, bi_ref, mi_ref, s_ref, keys_ref, vals_ref):
    fmo = fmo_ref[...]
    mentioned = jnp.max(fmo, axis=2) > 0.97
    mf = mentioned.astype(jnp.float32)
    il = lax.broadcasted_iota(jnp.int32, (_L, _L), 0)
    jl = lax.broadcasted_iota(jnp.int32, (_L, _L), 1)
    tri = (il <= jl).astype(jnp.float32)
    c = jnp.dot(mf, tri, preferred_element_type=jnp.float32)
    maskf = (c > 0).astype(jnp.float32)
    new0 = (c == 0).astype(jnp.float32)
    ml = likes_ref[...] * maskf
    maxv = jnp.max(ml, axis=1, keepdims=True)
    iota_l = lax.broadcasted_iota(jnp.int32, (_P, _L), 1)
    cand = jnp.where(ml == maxv, iota_l, jnp.int32(_L))
    idx = jnp.min(cand, axis=1, keepdims=True)
    onehot = (iota_l == idx).astype(jnp.float32)
    w = jnp.sum(onehot * new0, axis=1, keepdims=True)

    bi = bi_ref[...]
    mi = mi_ref[...]
    keys = bi * _N + mi
    keys_row = lax.dot_general(
        jnp.ones((_P, 1), jnp.float32), keys.astype(jnp.float32),
        (((1,), (1,)), ((), ())), preferred_element_type=jnp.float32)
    pcol = lax.broadcasted_iota(jnp.int32, (_P, _P), 0)
    prow = lax.broadcasted_iota(jnp.int32, (_P, _P), 1)
    eq = keys.astype(jnp.float32) == keys_row
    loser = jnp.sum(jnp.where(eq & (prow > pcol), 1.0, 0.0), axis=1,
                    keepdims=True) > 0
    winner = ~loser

    wm = jnp.where(winner, maxv, 0.0)
    ib = lax.broadcasted_iota(jnp.int32, (_P, _B), 1)
    s_ref[...] = jnp.where(bi == ib, wm, 0.0)
    keys_ref[...] = keys
    code = 1 + idx + 32 * (w > 0).astype(jnp.int32)
    vals_ref[...] = jnp.where(winner, code, 0)


def _dense_kernel(st_ref, g_ref, b1_ref, w2_ref, b2_ref, code_ref, out_ref,
                  h_ref):
    @pl.when(pl.program_id(0) == 0)
    def _():
        hp = lax.dot_general(st_ref[...], g_ref[...], (((0,), (0,)), ((), ())),
                             preferred_element_type=jnp.float32)
        h_ref[...] = jnp.maximum(hp + b1_ref[...], 0.0)
    h = h_ref[...]
    z = jnp.dot(h, w2_ref[...], preferred_element_type=jnp.float32)
    z = z + b2_ref[...]
    rec = 1.0 / (1.0 + jnp.exp(-z))
    code = code_ref[...]
    present = code > 0
    t = code - 1
    lsel = jnp.where(present, t & 31, 0)
    wv = jnp.where(present, (t >= 32).astype(jnp.float32), 1.0)
    recwv = rec * wv
    li = lax.broadcasted_iota(jnp.int32, (_B, _L, _NBLK), 1)
    out_ref[...] = jnp.where(li == lsel[:, None, :], recwv[:, None, :], 0.0)


def _stats(movie_likes, fmo, batch_indices, movie_indices):
    bi = batch_indices.astype(jnp.int32).reshape(_P, 1)
    mi = movie_indices.astype(jnp.int32).reshape(_P, 1)
    return pl.pallas_call(
        _stats_kernel,
        out_shape=(
            jax.ShapeDtypeStruct((_P, _B), jnp.float32),
            jax.ShapeDtypeStruct((_P, 1), jnp.int32),
            jax.ShapeDtypeStruct((_P, 1), jnp.int32),
        ),
    )(movie_likes, fmo, bi, mi)


def _dense(s_t, g, b1, w2, b2, codemap):
    grid = (pl.cdiv(_N, _NBLK),)
    return pl.pallas_call(
        _dense_kernel,
        grid=grid,
        in_specs=[
            pl.BlockSpec((_P, _B), lambda j: (0, 0)),
            pl.BlockSpec((_P, _H), lambda j: (0, 0)),
            pl.BlockSpec((1, _H), lambda j: (0, 0)),
            pl.BlockSpec((_H, _NBLK), lambda j: (0, j)),
            pl.BlockSpec((1, _NBLK), lambda j: (0, j)),
            pl.BlockSpec((_B, _NBLK), lambda j: (0, j)),
        ],
        out_specs=pl.BlockSpec((_B, _L, _NBLK), lambda j: (0, 0, j)),
        out_shape=jax.ShapeDtypeStruct((_B, _L, _N), jnp.float32),
        scratch_shapes=[pltpu.VMEM((_B, _H), jnp.float32)],
    )(s_t, g, b1.reshape(1, _H), w2, b2.reshape(1, _N), codemap)


_NW = 32
_PPW = _P // _NW
_CMW = _B * _N // _NW


def _sc_kernel(w1_hbm, mi_hbm, keys_hbm, vals_hbm, g_hbm, cm_hbm,
               idx_v, rows_v, keys_v, vals_v, sidx_v, sval_v, zbuf_v, cm_sh,
               sem):
    cid = lax.axis_index("c")
    sid = lax.axis_index("s")
    wid = sid * 2 + cid
    base = wid * _PPW
    pltpu.sync_copy(mi_hbm.at[pl.ds(base, _PPW)], idx_v)
    pltpu.async_copy(w1_hbm.at[idx_v], rows_v, sem).wait()
    pltpu.sync_copy(rows_v, g_hbm.at[pl.ds(base, _PPW)])
    cbase = wid * _CMW
    sbase = sid * _CMW

    def zero_body(i, carry):
        zbuf_v[pl.ds(i * 16, 16)] = jnp.zeros((16,), jnp.int32)
        return carry

    lax.fori_loop(0, _CMW // 16, zero_body, 0, unroll=8)
    pltpu.sync_copy(zbuf_v, cm_sh.at[pl.ds(sbase, _CMW)])
    pltpu.sync_copy(keys_hbm, keys_v)
    pltpu.sync_copy(vals_hbm, vals_v)

    for i in range(_P // 16):
        k = keys_v[pl.ds(i * 16, 16)]
        v = vals_v[pl.ds(i * 16, 16)]
        local = k - cbase
        m = (local >= 0) & (local < _CMW) & (v > 0)
        sidx_v[i // 8, pl.ds((i % 8) * 16, 16)] = jnp.where(m, local + sbase, 0)
        sval_v[i // 8, pl.ds((i % 8) * 16, 16)] = jnp.where(m, v, 0)
    for j in range(_P // 128):
        pltpu.sync_copy(sval_v.at[j], cm_sh.at[sidx_v.at[j]], add=True)
    pltpu.sync_copy(cm_sh.at[pl.ds(sbase, _CMW)], zbuf_v)
    pltpu.sync_copy(zbuf_v, cm_hbm.at[pl.ds(cbase, _CMW)])


def _sparse_mid(w1, movie_indices, keys, vals):
    call = functools.partial(
        pl.kernel,
        out_type=(
            jax.ShapeDtypeStruct((_P, _H), jnp.float32),
            jax.ShapeDtypeStruct((_B * _N,), jnp.int32),
        ),
        mesh=plsc.VectorSubcoreMesh(core_axis_name="c", subcore_axis_name="s"),
        scratch_types=[
            pltpu.VMEM((_PPW,), jnp.int32),
            pltpu.VMEM((_PPW, _H), jnp.float32),
            pltpu.VMEM((_P,), jnp.int32),
            pltpu.VMEM((_P,), jnp.int32),
            pltpu.VMEM((_P // 128, 128), jnp.int32),
            pltpu.VMEM((_P // 128, 128), jnp.int32),
            pltpu.VMEM((_CMW,), jnp.int32),
            pltpu.VMEM_SHARED((16 * _CMW,), jnp.int32),
            pltpu.SemaphoreType.DMA,
        ],
    )(_sc_kernel)
    return call(w1, movie_indices.astype(jnp.int32), keys.reshape(_P),
                vals.reshape(_P))


def kernel(movie_likes, flattened_movie_occurrences, batch_indices,
           movie_indices, W1, b1, W2, b2):
    s_t, keys, vals = _stats(movie_likes, flattened_movie_occurrences,
                             batch_indices, movie_indices)
    g, codemap = _sparse_mid(W1, movie_indices, keys, vals)
    out = _dense(s_t, g, b1, W2, b2, codemap.reshape(_B, _N))
    return out


# speedup vs baseline: 1.0068x; 1.0068x over previous
import functools

import jax
import jax.numpy as jnp
from jax import lax
from jax.experimental import pallas as pl
from jax.experimental.pallas import tpu as pltpu
from jax.experimental.pallas import tpu_sc as plsc

_B, _L, _N, _H, _P = 32, 20, 50000, 512, 256
_NBLK = 4096


def _stats_kernel(likes_ref, fmo_ref, bi_ref, mi_ref, s_ref, keys_ref, vals_ref):
    fmo = fmo_ref[...]
    mentioned = jnp.max(fmo, axis=2) > 0.97
    mf = mentioned.astype(jnp.float32)
    il = lax.broadcasted_iota(jnp.int32, (_L, _L), 0)
    jl = lax.broadcasted_iota(jnp.int32, (_L, _L), 1)
    tri = (il <= jl).astype(jnp.float32)
    c = jnp.dot(mf, tri, preferred_element_type=jnp.float32)
    maskf = (c > 0).astype(jnp.float32)
    new0 = (c == 0).astype(jnp.float32)
    ml = likes_ref[...] * maskf
    maxv = jnp.max(ml, axis=1, keepdims=True)
    iota_l = lax.broadcasted_iota(jnp.int32, (_P, _L), 1)
    cand = jnp.where(ml == maxv, iota_l, jnp.int32(_L))
    idx = jnp.min(cand, axis=1, keepdims=True)
    onehot = (iota_l == idx).astype(jnp.float32)
    w = jnp.sum(onehot * new0, axis=1, keepdims=True)

    bi = bi_ref[...]
    mi = mi_ref[...]
    keys = bi * _N + mi
    keys_row = lax.dot_general(
        jnp.ones((_P, 1), jnp.float32), keys.astype(jnp.float32),
        (((1,), (1,)), ((), ())), preferred_element_type=jnp.float32)
    pcol = lax.broadcasted_iota(jnp.int32, (_P, _P), 0)
    prow = lax.broadcasted_iota(jnp.int32, (_P, _P), 1)
    eq = keys.astype(jnp.float32) == keys_row
    loser = jnp.sum(jnp.where(eq & (prow > pcol), 1.0, 0.0), axis=1,
                    keepdims=True) > 0
    winner = ~loser

    wm = jnp.where(winner, maxv, 0.0)
    ib = lax.broadcasted_iota(jnp.int32, (_P, _B), 1)
    s_ref[...] = jnp.where(bi == ib, wm, 0.0)
    keys_ref[...] = keys
    code = 1 + idx + 32 * (w > 0).astype(jnp.int32)
    vals_ref[...] = jnp.where(winner, code, 0)


def _dense_kernel(st_ref, g_ref, b1_ref, w2_ref, b2_ref, code_ref, out_ref,
                  h_ref):
    @pl.when(pl.program_id(0) == 0)
    def _():
        hp = lax.dot_general(st_ref[...], g_ref[...], (((0,), (0,)), ((), ())),
                             preferred_element_type=jnp.float32)
        h_ref[...] = jnp.maximum(hp + b1_ref[...], 0.0)
    h = h_ref[...]
    z = jnp.dot(h, w2_ref[...], preferred_element_type=jnp.float32)
    z = z + b2_ref[...]
    rec = 1.0 / (1.0 + jnp.exp(-z))
    code = code_ref[...]
    present = code > 0
    t = code - 1
    lsel = jnp.where(present, t & 31, 0)
    wv = jnp.where(present, (t >= 32).astype(jnp.float32), 1.0)
    recwv = rec * wv
    li = lax.broadcasted_iota(jnp.int32, (_B, _L, _NBLK), 1)
    out_ref[...] = jnp.where(li == lsel[:, None, :], recwv[:, None, :], 0.0)


def _stats(movie_likes, fmo, batch_indices, movie_indices):
    bi = batch_indices.astype(jnp.int32).reshape(_P, 1)
    mi = movie_indices.astype(jnp.int32).reshape(_P, 1)
    return pl.pallas_call(
        _stats_kernel,
        out_shape=(
            jax.ShapeDtypeStruct((_P, _B), jnp.float32),
            jax.ShapeDtypeStruct((_P, 1), jnp.int32),
            jax.ShapeDtypeStruct((_P, 1), jnp.int32),
        ),
    )(movie_likes, fmo, bi, mi)


def _dense(s_t, g, b1, w2, b2, codemap):
    grid = (pl.cdiv(_N, _NBLK),)
    return pl.pallas_call(
        _dense_kernel,
        grid=grid,
        in_specs=[
            pl.BlockSpec((_P, _B), lambda j: (0, 0)),
            pl.BlockSpec((_P, _H), lambda j: (0, 0)),
            pl.BlockSpec((1, _H), lambda j: (0, 0)),
            pl.BlockSpec((_H, _NBLK), lambda j: (0, j)),
            pl.BlockSpec((1, _NBLK), lambda j: (0, j)),
            pl.BlockSpec((_B, _NBLK), lambda j: (0, j)),
        ],
        out_specs=pl.BlockSpec((_B, _L, _NBLK), lambda j: (0, 0, j)),
        out_shape=jax.ShapeDtypeStruct((_B, _L, _N), jnp.float32),
        scratch_shapes=[pltpu.VMEM((_B, _H), jnp.float32)],
    )(s_t, g, b1.reshape(1, _H), w2, b2.reshape(1, _N), codemap)


_NW = 32
_PPW = _P // _NW
_CMW = _B * _N // _NW


def _sc_kernel(w1_hbm, mi_hbm, keys_hbm, vals_hbm, g_hbm, cm_hbm,
               idx_v, rows_v, keys_v, vals_v, sidx_v, sval_v, zbuf_v, cm_sh,
               sem):
    cid = lax.axis_index("c")
    sid = lax.axis_index("s")
    wid = sid * 2 + cid
    base = wid * _PPW
    pltpu.sync_copy(mi_hbm.at[pl.ds(base, _PPW)], idx_v)
    pltpu.async_copy(w1_hbm.at[idx_v], rows_v, sem).wait()
    pltpu.sync_copy(rows_v, g_hbm.at[pl.ds(base, _PPW)])
    cbase = wid * _CMW
    sbase = sid * _CMW

    def zero_body(i, carry):
        zbuf_v[pl.ds(i * 16, 16)] = jnp.zeros((16,), jnp.int32)
        return carry

    lax.fori_loop(0, _CMW // 16, zero_body, 0, unroll=8)
    pltpu.sync_copy(zbuf_v, cm_sh.at[pl.ds(sbase, _CMW)])
    pltpu.sync_copy(keys_hbm, keys_v)
    pltpu.sync_copy(vals_hbm, vals_v)

    for i in range(_P // 16):
        k = keys_v[pl.ds(i * 16, 16)]
        v = vals_v[pl.ds(i * 16, 16)]
        local = k - cbase
        m = (local >= 0) & (local < _CMW) & (v > 0)
        sidx_v[i // 8, pl.ds((i % 8) * 16, 16)] = jnp.where(m, local + sbase, 0)
        sval_v[i // 8, pl.ds((i % 8) * 16, 16)] = jnp.where(m, v, 0)
    for j in range(_P // 128):
        pltpu.sync_copy(sval_v.at[j], cm_sh.at[sidx_v.at[j]], add=True)
    pltpu.sync_copy(cm_sh.at[pl.ds(sbase, _CMW)], zbuf_v)
    pltpu.sync_copy(zbuf_v, cm_hbm.at[pl.ds(cbase, _CMW)])


def _sparse_mid(w1, movie_indices, keys, vals):
    call = functools.partial(
        pl.kernel,
        out_type=(
            jax.ShapeDtypeStruct((_P, _H), jnp.float32),
            jax.ShapeDtypeStruct((_B * _N,), jnp.int32),
        ),
        mesh=plsc.VectorSubcoreMesh(core_axis_name="c", subcore_axis_name="s"),
        scratch_types=[
            pltpu.VMEM((_PPW,), jnp.int32),
            pltpu.VMEM((_PPW, _H), jnp.float32),
            pltpu.VMEM((_P,), jnp.int32),
            pltpu.VMEM((_P,), jnp.int32),
            pltpu.VMEM((_P // 128, 128), jnp.int32),
            pltpu.VMEM((_P // 128, 128), jnp.int32),
            pltpu.VMEM((_CMW,), jnp.int32),
            pltpu.VMEM_SHARED((16 * _CMW,), jnp.int32),
            pltpu.SemaphoreType.DMA,
        ],
    )(_sc_kernel)
    return call(w1, movie_indices.astype(jnp.int32), keys.reshape(_P),
                vals.reshape(_P))


def kernel(movie_likes, flattened_movie_occurrences, batch_indices,
           movie_indices, W1, b1, W2, b2):
    s_t, keys, vals = _stats(movie_likes, flattened_movie_occurrences,
                             batch_indices, movie_indices)
    g, codemap = _sparse_mid(W1, movie_indices, keys, vals)
    out = _dense(s_t, g, b1, W2, b2, codemap.reshape(_B, _N))
    return out


# speedup vs baseline: 2.3497x; 2.3338x over previous
import functools

import jax
import jax.numpy as jnp
from jax import lax
from jax.experimental import pallas as pl
from jax.experimental.pallas import tpu as pltpu
from jax.experimental.pallas import tpu_sc as plsc

_B, _L, _N, _H, _P = 32, 20, 50000, 512, 256
_NBLK = 2048


def _stats_kernel(likes_ref, fmo_ref, bi_ref, mi_ref, s_ref, keys_ref, vals_ref):
    fmo = fmo_ref[...]
    mentioned = jnp.max(fmo, axis=2) > 0.97
    mf = mentioned.astype(jnp.float32)
    il = lax.broadcasted_iota(jnp.int32, (_L, _L), 0)
    jl = lax.broadcasted_iota(jnp.int32, (_L, _L), 1)
    tri = (il <= jl).astype(jnp.float32)
    c = jnp.dot(mf, tri, preferred_element_type=jnp.float32)
    maskf = (c > 0).astype(jnp.float32)
    new0 = (c == 0).astype(jnp.float32)
    ml = likes_ref[...] * maskf
    maxv = jnp.max(ml, axis=1, keepdims=True)
    iota_l = lax.broadcasted_iota(jnp.int32, (_P, _L), 1)
    cand = jnp.where(ml == maxv, iota_l, jnp.int32(_L))
    idx = jnp.min(cand, axis=1, keepdims=True)
    onehot = (iota_l == idx).astype(jnp.float32)
    w = jnp.sum(onehot * new0, axis=1, keepdims=True)

    bi = bi_ref[...]
    mi = mi_ref[...]
    keys = bi * _N + mi
    keys_row = lax.dot_general(
        jnp.ones((_P, 1), jnp.float32), keys.astype(jnp.float32),
        (((1,), (1,)), ((), ())), preferred_element_type=jnp.float32)
    pcol = lax.broadcasted_iota(jnp.int32, (_P, _P), 0)
    prow = lax.broadcasted_iota(jnp.int32, (_P, _P), 1)
    eq = keys.astype(jnp.float32) == keys_row
    loser = jnp.sum(jnp.where(eq & (prow > pcol), 1.0, 0.0), axis=1,
                    keepdims=True) > 0
    winner = ~loser

    wm = jnp.where(winner, maxv, 0.0)
    ib = lax.broadcasted_iota(jnp.int32, (_P, _B), 1)
    s_ref[...] = jnp.where(bi == ib, wm, 0.0)
    keys_ref[...] = keys
    code = 1 + idx + 32 * (w > 0).astype(jnp.int32)
    vals_ref[...] = jnp.where(winner, code, 0)


def _dense_kernel(st_ref, g_ref, b1_ref, w2t_ref, b2_ref, code_ref, out_ref,
                  h_ref):
    @pl.when(pl.program_id(0) == 0)
    def _():
        hp = lax.dot_general(st_ref[...], g_ref[...], (((0,), (0,)), ((), ())),
                             preferred_element_type=jnp.float32)
        h_ref[...] = jnp.maximum(hp + b1_ref[...], 0.0)
    h = h_ref[...]
    z = lax.dot_general(h, w2t_ref[...], (((1,), (1,)), ((), ())),
                        preferred_element_type=jnp.float32)
    z = z + b2_ref[...]
    rec = 1.0 / (1.0 + jnp.exp(-z))
    code = code_ref[...]
    present = code > 0
    t = code - 1
    lsel = jnp.where(present, t & 31, 0)
    wv = jnp.where(present, (t >= 32).astype(jnp.float32), 1.0)
    recwv = rec * wv
    li = lax.broadcasted_iota(jnp.int32, (_L, _B, _NBLK), 0)
    out_ref[...] = jnp.where(li == lsel[None, :, :], recwv[None, :, :], 0.0)


def _stats(movie_likes, fmo, batch_indices, movie_indices):
    bi = batch_indices.astype(jnp.int32).reshape(_P, 1)
    mi = movie_indices.astype(jnp.int32).reshape(_P, 1)
    return pl.pallas_call(
        _stats_kernel,
        out_shape=(
            jax.ShapeDtypeStruct((_P, _B), jnp.float32),
            jax.ShapeDtypeStruct((_P, 1), jnp.int32),
            jax.ShapeDtypeStruct((_P, 1), jnp.int32),
        ),
    )(movie_likes, fmo, bi, mi)


def _dense(s_t, g, b1, w2t, b2, codemap):
    grid = (pl.cdiv(_N, _NBLK),)
    return pl.pallas_call(
        _dense_kernel,
        grid=grid,
        in_specs=[
            pl.BlockSpec((_P, _B), lambda j: (0, 0)),
            pl.BlockSpec((_P, _H), lambda j: (0, 0)),
            pl.BlockSpec((1, _H), lambda j: (0, 0)),
            pl.BlockSpec((_NBLK, _H), lambda j: (j, 0)),
            pl.BlockSpec((1, _NBLK), lambda j: (0, j)),
            pl.BlockSpec((_B, _NBLK), lambda j: (0, j)),
        ],
        out_specs=pl.BlockSpec((_L, _B, _NBLK), lambda j: (0, 0, j)),
        out_shape=jax.ShapeDtypeStruct((_L, _B, _N), jnp.float32),
        scratch_shapes=[pltpu.VMEM((_B, _H), jnp.float32)],
    )(s_t, g, b1.reshape(1, _H), w2t, b2.reshape(1, _N), codemap)


_NW = 32
_PPW = _P // _NW
_CMW = _B * _N // _NW


def _sc_kernel(w1_hbm, mi_hbm, keys_hbm, vals_hbm, g_hbm, cm_hbm,
               idx_v, rows_v, keys_v, vals_v, sidx_v, sval_v, zbuf_v, cm_sh,
               sem):
    cid = lax.axis_index("c")
    sid = lax.axis_index("s")
    wid = sid * 2 + cid
    base = wid * _PPW
    pltpu.sync_copy(mi_hbm.at[pl.ds(base, _PPW)], idx_v)
    pltpu.async_copy(w1_hbm.at[idx_v], rows_v, sem).wait()
    pltpu.sync_copy(rows_v, g_hbm.at[pl.ds(base, _PPW)])
    cbase = wid * _CMW
    sbase = sid * _CMW

    def zero_body(i, carry):
        zbuf_v[pl.ds(i * 16, 16)] = jnp.zeros((16,), jnp.int32)
        return carry

    lax.fori_loop(0, _CMW // 16, zero_body, 0, unroll=8)
    pltpu.sync_copy(zbuf_v, cm_sh.at[pl.ds(sbase, _CMW)])
    pltpu.sync_copy(keys_hbm, keys_v)
    pltpu.sync_copy(vals_hbm, vals_v)

    for i in range(_P // 16):
        k = keys_v[pl.ds(i * 16, 16)]
        v = vals_v[pl.ds(i * 16, 16)]
        local = k - cbase
        m = (local >= 0) & (local < _CMW) & (v > 0)
        sidx_v[i // 8, pl.ds((i % 8) * 16, 16)] = jnp.where(m, local + sbase, 0)
        sval_v[i // 8, pl.ds((i % 8) * 16, 16)] = jnp.where(m, v, 0)
    for j in range(_P // 128):
        pltpu.sync_copy(sval_v.at[j], cm_sh.at[sidx_v.at[j]], add=True)
    pltpu.sync_copy(cm_sh.at[pl.ds(sbase, _CMW)], zbuf_v)
    pltpu.sync_copy(zbuf_v, cm_hbm.at[pl.ds(cbase, _CMW)])


def _sparse_mid(w1, movie_indices, keys, vals):
    call = functools.partial(
        pl.kernel,
        out_type=(
            jax.ShapeDtypeStruct((_P, _H), jnp.float32),
            jax.ShapeDtypeStruct((_B * _N,), jnp.int32),
        ),
        mesh=plsc.VectorSubcoreMesh(core_axis_name="c", subcore_axis_name="s"),
        scratch_types=[
            pltpu.VMEM((_PPW,), jnp.int32),
            pltpu.VMEM((_PPW, _H), jnp.float32),
            pltpu.VMEM((_P,), jnp.int32),
            pltpu.VMEM((_P,), jnp.int32),
            pltpu.VMEM((_P // 128, 128), jnp.int32),
            pltpu.VMEM((_P // 128, 128), jnp.int32),
            pltpu.VMEM((_CMW,), jnp.int32),
            pltpu.VMEM_SHARED((16 * _CMW,), jnp.int32),
            pltpu.SemaphoreType.DMA,
        ],
    )(_sc_kernel)
    return call(w1, movie_indices.astype(jnp.int32), keys.reshape(_P),
                vals.reshape(_P))


def kernel(movie_likes, flattened_movie_occurrences, batch_indices,
           movie_indices, W1, b1, W2, b2):
    s_t, keys, vals = _stats(movie_likes, flattened_movie_occurrences,
                             batch_indices, movie_indices)
    g, codemap = _sparse_mid(W1, movie_indices, keys, vals)
    w2t = jnp.swapaxes(W2, 0, 1)
    out = _dense(s_t, g, b1, w2t, b2, codemap.reshape(_B, _N))
    return jnp.transpose(out, (1, 0, 2))
